# baseline (speedup 1.0000x reference)
import functools

import jax
import jax.numpy as jnp
from jax import lax
from jax.experimental import pallas as pl
from jax.experimental.pallas import tpu as pltpu
from jax.experimental.pallas import tpu_sc as plsc

B = 4096
T = 200
E = 64
BT = B * T
NC = 2
NS = 16
NW = NC * NS
CHUNK = 128
SEQ_PER_W = BT // NW
SEQ_CHUNKS = SEQ_PER_W // CHUNK
TGT_PER_W = B // NW

RB = 16
SEQ_ROWS = RB * T
SEQ_GRID = BT // SEQ_ROWS
TGT_ROWS = 1024
TGT_GRID = B // TGT_ROWS


def _sc_gather_body(item_tab, cate_tab,
                    hist_i, hist_c, tgt_i, tgt_c,
                    gcat_out, tcat_out,
                    idx_i, idx_c, bi, bc, sem_a, sem_b):
    wid = lax.axis_index("s") * NC + lax.axis_index("c")
    pltpu.sync_copy(hist_i.at[pl.ds(wid * SEQ_CHUNKS, SEQ_CHUNKS)], idx_i)
    pltpu.sync_copy(hist_c.at[pl.ds(wid * SEQ_CHUNKS, SEQ_CHUNKS)], idx_c)
    base = wid * SEQ_PER_W

    def body(j, carry):
        ca = pltpu.async_copy(item_tab.at[idx_i.at[j]], bi, sem_a)
        cb = pltpu.async_copy(cate_tab.at[idx_c.at[j]], bc, sem_b)
        ca.wait()
        pltpu.sync_copy(bi.at[pl.ds(0, CHUNK), pl.ds(E, E)],
                        gcat_out.at[pl.ds(base + j * CHUNK, CHUNK),
                                    pl.ds(0, E)])
        cb.wait()
        pltpu.sync_copy(bc.at[pl.ds(0, CHUNK), pl.ds(E, E)],
                        gcat_out.at[pl.ds(base + j * CHUNK, CHUNK),
                                    pl.ds(E, E)])
        return carry

    lax.fori_loop(0, SEQ_CHUNKS, body, 0)

    pltpu.sync_copy(tgt_i.at[pl.ds(wid, 1)], idx_i.at[pl.ds(0, 1)])
    pltpu.sync_copy(tgt_c.at[pl.ds(wid, 1)], idx_c.at[pl.ds(0, 1)])
    ca = pltpu.async_copy(item_tab.at[idx_i.at[0]], bi, sem_a)
    cb = pltpu.async_copy(cate_tab.at[idx_c.at[0]], bc, sem_b)
    ca.wait()
    pltpu.sync_copy(bi.at[pl.ds(0, CHUNK), pl.ds(0, E)],
                    tcat_out.at[pl.ds(wid * TGT_PER_W, TGT_PER_W),
                                pl.ds(0, E)])
    cb.wait()
    pltpu.sync_copy(bc.at[pl.ds(0, CHUNK), pl.ds(0, E)],
                    tcat_out.at[pl.ds(wid * TGT_PER_W, TGT_PER_W),
                                pl.ds(E, E)])


_sc_gather = functools.partial(
    pl.kernel,
    mesh=plsc.VectorSubcoreMesh(core_axis_name="c", subcore_axis_name="s",
                                num_cores=NC, num_subcores=NS),
    out_type=[
        jax.ShapeDtypeStruct((BT, 2 * E), jnp.float32),
        jax.ShapeDtypeStruct((B, 2 * E), jnp.float32),
    ],
    scratch_types=[
        pltpu.VMEM((SEQ_CHUNKS, CHUNK), jnp.int32),
        pltpu.VMEM((SEQ_CHUNKS, CHUNK), jnp.int32),
        pltpu.VMEM((CHUNK, 2 * E), jnp.float32),
        pltpu.VMEM((CHUNK, 2 * E), jnp.float32),
        pltpu.SemaphoreType.DMA,
        pltpu.SemaphoreType.DMA,
    ],
    compiler_params=pltpu.CompilerParams(use_tc_tiling_on_sc=False),
)(_sc_gather_body)


def _seq_body(gcat_ref, wp_ref, pos_ref, out_ref):
    out_ref[...] = (
        jnp.dot(gcat_ref[...], wp_ref[...], preferred_element_type=jnp.float32)
        + pos_ref[...]
    )


def _seq_project(gcat, wp, pos_tiled):
    return pl.pallas_call(
        _seq_body,
        grid=(SEQ_GRID,),
        in_specs=[
            pl.BlockSpec((SEQ_ROWS, 2 * E), lambda i: (i, 0)),
            pl.BlockSpec((2 * E, 2 * E), lambda i: (0, 0)),
            pl.BlockSpec((SEQ_ROWS, 2 * E), lambda i: (0, 0)),
        ],
        out_specs=pl.BlockSpec((SEQ_ROWS, 2 * E), lambda i: (i, 0)),
        out_shape=jax.ShapeDtypeStruct((BT, 2 * E), jnp.float32),
        compiler_params=pltpu.CompilerParams(
            dimension_semantics=("parallel",)),
    )(gcat, wp, pos_tiled)


def _tgt_body(tcat_ref, w1_ref, b1_ref, w2_ref, b2_ref, out_ref):
    h = (jnp.dot(tcat_ref[...], w1_ref[...], preferred_element_type=jnp.float32)
         + b1_ref[...])
    h = jnp.maximum(h, 0.0)
    out_ref[...] = (
        jnp.dot(h, w2_ref[...], preferred_element_type=jnp.float32) + b2_ref[...]
    )


def _tgt_mlp(tcat, w1, b1, w2, b2):
    return pl.pallas_call(
        _tgt_body,
        grid=(TGT_GRID,),
        in_specs=[
            pl.BlockSpec((TGT_ROWS, 2 * E), lambda i: (i, 0)),
            pl.BlockSpec((2 * E, 256), lambda i: (0, 0)),
            pl.BlockSpec((1, 256), lambda i: (0, 0)),
            pl.BlockSpec((256, 2 * E), lambda i: (0, 0)),
            pl.BlockSpec((1, 2 * E), lambda i: (0, 0)),
        ],
        out_specs=pl.BlockSpec((TGT_ROWS, 2 * E), lambda i: (i, 0)),
        out_shape=jax.ShapeDtypeStruct((B, 2 * E), jnp.float32),
        compiler_params=pltpu.CompilerParams(
            dimension_semantics=("parallel",)),
    )(tcat, w1, b1, w2, b2)


def kernel(target_item_id, target_cate_id, hist_item_ids, hist_cate_ids,
           tgt_item_table, tgt_cate_table, W1, b1, W2, b2,
           seq_item_table, seq_cate_table, Wp, pos_table):
    hist_i = hist_item_ids.astype(jnp.int32).reshape(BT // CHUNK, CHUNK)
    hist_c = hist_cate_ids.astype(jnp.int32).reshape(BT // CHUNK, CHUNK)
    tgt_i = target_item_id.astype(jnp.int32).reshape(NW, TGT_PER_W)
    tgt_c = target_cate_id.astype(jnp.int32).reshape(NW, TGT_PER_W)

    item_tab = jnp.concatenate([tgt_item_table, seq_item_table], axis=1)
    cate_tab = jnp.concatenate([tgt_cate_table, seq_cate_table], axis=1)
    gcat, tcat = _sc_gather(item_tab, cate_tab,
                            hist_i, hist_c, tgt_i, tgt_c)

    pos_tiled = jnp.tile(pos_table, (RB, 1))
    seq = _seq_project(gcat, Wp, pos_tiled).reshape(B, T, 2 * E)

    proj = _tgt_mlp(tcat, W1, b1.reshape(1, -1), W2, b2.reshape(1, -1))
    query = proj.reshape(B, 4, 32)
    return (query, seq)


# speedup vs baseline: 1.1356x; 1.1356x over previous
import functools

import jax
import jax.numpy as jnp
from jax import lax
from jax.experimental import pallas as pl
from jax.experimental.pallas import tpu as pltpu
from jax.experimental.pallas import tpu_sc as plsc

B = 4096
T = 200
E = 64
VI = 1000000
VC = 100000
BT = B * T
NC = 2
NS = 16
NW = NC * NS
CHUNK = 128
SEQ_PER_W = BT // NW
SEQ_CHUNKS = SEQ_PER_W // CHUNK
TGT_PER_W = B // NW

RB = 16
SEQ_ROWS = RB * T
SEQ_GRID = BT // SEQ_ROWS
TGT_ROWS = 1024
TGT_GRID = B // TGT_ROWS


def _sc_gather_body(item_tab, cate_tab,
                    hist_i, hist_c, tgt_i, tgt_c,
                    gcat_out, tcat_out,
                    idx_i, idx_c, bi, bc, bi2, bc2,
                    sem_a, sem_b, sem_a2, sem_b2):
    wid = lax.axis_index("s") * NC + lax.axis_index("c")
    pltpu.sync_copy(hist_i.at[pl.ds(wid * SEQ_CHUNKS, SEQ_CHUNKS)], idx_i)
    pltpu.sync_copy(hist_c.at[pl.ds(wid * SEQ_CHUNKS, SEQ_CHUNKS)], idx_c)
    base = wid * SEQ_PER_W

    def fire(j, slot_i, slot_c, sa, sb):
        pltpu.async_copy(item_tab.at[idx_i.at[j]], slot_i, sa)
        pltpu.async_copy(cate_tab.at[idx_c.at[j]], slot_c, sb)

    def drain_and_store(j, slot_i, slot_c, sa, sb):
        pltpu.make_async_copy(item_tab.at[pl.ds(0, CHUNK)], slot_i, sa).wait()
        pltpu.sync_copy(slot_i.at[pl.ds(0, CHUNK), pl.ds(E, E)],
                        gcat_out.at[pl.ds(base + j * CHUNK, CHUNK),
                                    pl.ds(0, E)])
        pltpu.make_async_copy(cate_tab.at[pl.ds(0, CHUNK)], slot_c, sb).wait()
        pltpu.sync_copy(slot_c.at[pl.ds(0, CHUNK), pl.ds(E, E)],
                        gcat_out.at[pl.ds(base + j * CHUNK, CHUNK),
                                    pl.ds(E, E)])

    fire(0, bi, bc, sem_a, sem_b)

    def body(t, carry):
        j0 = 2 * t
        fire(j0 + 1, bi2, bc2, sem_a2, sem_b2)
        drain_and_store(j0, bi, bc, sem_a, sem_b)

        @pl.when(t < SEQ_CHUNKS // 2 - 1)
        def _():
            fire(j0 + 2, bi, bc, sem_a, sem_b)

        drain_and_store(j0 + 1, bi2, bc2, sem_a2, sem_b2)
        return carry

    lax.fori_loop(0, SEQ_CHUNKS // 2, body, 0)

    pltpu.sync_copy(tgt_i.at[pl.ds(wid, 1)], idx_i.at[pl.ds(0, 1)])
    pltpu.sync_copy(tgt_c.at[pl.ds(wid, 1)], idx_c.at[pl.ds(0, 1)])
    ca = pltpu.async_copy(item_tab.at[idx_i.at[0]], bi, sem_a)
    cb = pltpu.async_copy(cate_tab.at[idx_c.at[0]], bc, sem_b)
    ca.wait()
    pltpu.sync_copy(bi.at[pl.ds(0, CHUNK), pl.ds(0, E)],
                    tcat_out.at[pl.ds(wid * TGT_PER_W, TGT_PER_W),
                                pl.ds(0, E)])
    cb.wait()
    pltpu.sync_copy(bc.at[pl.ds(0, CHUNK), pl.ds(0, E)],
                    tcat_out.at[pl.ds(wid * TGT_PER_W, TGT_PER_W),
                                pl.ds(E, E)])


_sc_gather = functools.partial(
    pl.kernel,
    mesh=plsc.VectorSubcoreMesh(core_axis_name="c", subcore_axis_name="s",
                                num_cores=NC, num_subcores=NS),
    out_type=[
        jax.ShapeDtypeStruct((BT, 2 * E), jnp.float32),
        jax.ShapeDtypeStruct((B, 2 * E), jnp.float32),
    ],
    scratch_types=[
        pltpu.VMEM((SEQ_CHUNKS, CHUNK), jnp.int32),
        pltpu.VMEM((SEQ_CHUNKS, CHUNK), jnp.int32),
        pltpu.VMEM((CHUNK, 2 * E), jnp.float32),
        pltpu.VMEM((CHUNK, 2 * E), jnp.float32),
        pltpu.VMEM((CHUNK, 2 * E), jnp.float32),
        pltpu.VMEM((CHUNK, 2 * E), jnp.float32),
        pltpu.SemaphoreType.DMA,
        pltpu.SemaphoreType.DMA,
        pltpu.SemaphoreType.DMA,
        pltpu.SemaphoreType.DMA,
    ],
    compiler_params=pltpu.CompilerParams(use_tc_tiling_on_sc=False),
)(_sc_gather_body)


def _seq_body(gcat_ref, wp_ref, pos_ref, out_ref):
    out_ref[...] = (
        jnp.dot(gcat_ref[...], wp_ref[...], preferred_element_type=jnp.float32)
        + pos_ref[...]
    )


def _seq_project(gcat, wp, pos_tiled):
    return pl.pallas_call(
        _seq_body,
        grid=(SEQ_GRID,),
        in_specs=[
            pl.BlockSpec((SEQ_ROWS, 2 * E), lambda i: (i, 0)),
            pl.BlockSpec((2 * E, 2 * E), lambda i: (0, 0)),
            pl.BlockSpec((SEQ_ROWS, 2 * E), lambda i: (0, 0)),
        ],
        out_specs=pl.BlockSpec((SEQ_ROWS, 2 * E), lambda i: (i, 0)),
        out_shape=jax.ShapeDtypeStruct((BT, 2 * E), jnp.float32),
        compiler_params=pltpu.CompilerParams(
            dimension_semantics=("parallel",)),
    )(gcat, wp, pos_tiled)


def _tgt_body(tcat_ref, w1_ref, b1_ref, w2_ref, b2_ref, out_ref):
    h = (jnp.dot(tcat_ref[...], w1_ref[...], preferred_element_type=jnp.float32)
         + b1_ref[...])
    h = jnp.maximum(h, 0.0)
    out_ref[...] = (
        jnp.dot(h, w2_ref[...], preferred_element_type=jnp.float32) + b2_ref[...]
    )


def _tgt_mlp(tcat, w1, b1, w2, b2):
    return pl.pallas_call(
        _tgt_body,
        grid=(TGT_GRID,),
        in_specs=[
            pl.BlockSpec((TGT_ROWS, 2 * E), lambda i: (i, 0)),
            pl.BlockSpec((2 * E, 256), lambda i: (0, 0)),
            pl.BlockSpec((1, 256), lambda i: (0, 0)),
            pl.BlockSpec((256, 2 * E), lambda i: (0, 0)),
            pl.BlockSpec((1, 2 * E), lambda i: (0, 0)),
        ],
        out_specs=pl.BlockSpec((TGT_ROWS, 2 * E), lambda i: (i, 0)),
        out_shape=jax.ShapeDtypeStruct((B, 2 * E), jnp.float32),
        compiler_params=pltpu.CompilerParams(
            dimension_semantics=("parallel",)),
    )(tcat, w1, b1, w2, b2)


def kernel(target_item_id, target_cate_id, hist_item_ids, hist_cate_ids,
           tgt_item_table, tgt_cate_table, W1, b1, W2, b2,
           seq_item_table, seq_cate_table, Wp, pos_table):
    hist_i = hist_item_ids.astype(jnp.int32).reshape(BT // CHUNK, CHUNK)
    hist_c = hist_cate_ids.astype(jnp.int32).reshape(BT // CHUNK, CHUNK)
    tgt_i = target_item_id.astype(jnp.int32).reshape(NW, TGT_PER_W)
    tgt_c = target_cate_id.astype(jnp.int32).reshape(NW, TGT_PER_W)

    item_tab = jnp.concatenate([tgt_item_table[:VI], seq_item_table[:VI]],
                               axis=1)
    cate_tab = jnp.concatenate([tgt_cate_table[:VC], seq_cate_table[:VC]],
                               axis=1)
    gcat, tcat = _sc_gather(item_tab, cate_tab,
                            hist_i, hist_c, tgt_i, tgt_c)

    pos_tiled = jnp.tile(pos_table, (RB, 1))
    seq = _seq_project(gcat, Wp, pos_tiled).reshape(B, T, 2 * E)

    proj = _tgt_mlp(tcat, W1, b1.reshape(1, -1), W2, b2.reshape(1, -1))
    query = proj.reshape(B, 4, 32)
    return (query, seq)
